# Initial kernel scaffold; baseline (speedup 1.0000x reference)
import jax
import jax.numpy as jnp
from jax.experimental import pallas as pl


def kernel(user_emb, item_emb, gate_W, gate_b, attn_W, ui_edge_index, ss_edge_index):
    raise NotImplementedError("write your pallas kernel here")


# speedup vs baseline: 6.6871x; 6.6871x over previous
import functools

import jax
import jax.numpy as jnp
from jax import lax
from jax.experimental import pallas as pl
from jax.experimental.pallas import tpu as pltpu
from jax.experimental.pallas import tpu_sc as plsc

NU = 5000
NI = 5000
D = 128
E = 160000

NCORES = 2
NSUB = 16
NW = NCORES * NSUB
EPT = E // NW
CH = 128
NFULL = EPT // CH
TAIL = EPT - NFULL * CH

_mesh = lambda: plsc.VectorSubcoreMesh(core_axis_name="c", subcore_axis_name="s")


def _zero_vmem(ref, rows, width):
    @pl.loop(0, rows)
    def _(i):
        for k in range(width // 16):
            ref[i, pl.ds(16 * k, 16)] = jnp.zeros((16,), jnp.float32)


def _fill_ones(ref, rows, width):
    @pl.loop(0, rows)
    def _(i):
        for k in range(width // 16):
            ref[i, pl.ds(16 * k, 16)] = jnp.ones((16,), jnp.float32)


def _rows_sweep(sub, fn):
    fn(pl.multiple_of(sub * 312, 8), 312)
    @pl.when(sub == 0)
    def _():
        fn(4992, 8)


def _zero_rows(sub, acc, zbuf):
    for t in range(3):
        pltpu.sync_copy(
            zbuf.at[pl.ds(0, 104)],
            acc.at[pl.ds(pl.multiple_of(sub * 312 + t * 104, 8), 104)])
    @pl.when(sub == 0)
    def _():
        pltpu.sync_copy(zbuf.at[pl.ds(0, 8)], acc.at[pl.ds(4992, 8)])


def _sc_degrees(ss_s, ss_d, ui_u, ui_i, out_hbm,
                acc0, acc1, acc2, acc3, idx_v, idx_t, ones_v, zbuf):
    core = lax.axis_index("c")
    sub = lax.axis_index("s")
    g = core * NSUB + sub

    _zero_vmem(zbuf, 312, 16)
    _fill_ones(ones_v, CH, 16)
    accs = (acc0, acc1, acc2, acc3)
    for acc in accs:
        _rows_sweep(sub, lambda off, n, a=acc: pltpu.sync_copy(
            zbuf.at[pl.ds(0, n)], a.at[pl.ds(off, n)]))
    plsc.subcore_barrier()

    for (edges, acc) in ((ss_s, acc0), (ss_d, acc1),
                         (ui_u, acc2), (ui_i, acc3)):
        @pl.loop(0, NFULL)
        def _(j):
            base = pl.multiple_of(g * EPT + j * CH, 8)
            pltpu.sync_copy(edges.at[pl.ds(base, CH)], idx_v)
            pltpu.sync_copy(ones_v, acc.at[idx_v], add=True)
        if TAIL:
            base = pl.multiple_of(g * EPT + NFULL * CH, 8)
            pltpu.sync_copy(edges.at[pl.ds(base, TAIL)], idx_t)
            pltpu.sync_copy(ones_v.at[pl.ds(0, TAIL)], acc.at[idx_t], add=True)

    plsc.subcore_barrier()
    for k, acc in enumerate(accs):
        _rows_sweep(sub, lambda off, n, a=acc, kk=k: pltpu.sync_copy(
            a.at[pl.ds(off, n)], out_hbm.at[core, kk, pl.ds(off, n)]))


def _degrees(ss_s, ss_d, ui_u, ui_i):
    fn = pl.kernel(
        _sc_degrees,
        out_type=jax.ShapeDtypeStruct((NCORES, 4, NU, 16), jnp.float32),
        mesh=_mesh(),
        scratch_types=[
            pltpu.VMEM_SHARED((NU, 16), jnp.float32),
            pltpu.VMEM_SHARED((NU, 16), jnp.float32),
            pltpu.VMEM_SHARED((NU, 16), jnp.float32),
            pltpu.VMEM_SHARED((NU, 16), jnp.float32),
            pltpu.VMEM((CH,), jnp.int32),
            pltpu.VMEM((TAIL,), jnp.int32),
            pltpu.VMEM((CH, 16), jnp.float32),
            pltpu.VMEM((312, 16), jnp.float32),
        ],
    )
    return fn(ss_s, ss_d, ui_u, ui_i)


def _sc_spmm_ss(x0, x1, x2, ss_s, ss_d, out_hbm,
                acc, idx_s, idx_d, idx_st, idx_dt,
                rows_v, rows_t, zbuf, sem):
    core = lax.axis_index("c")
    sub = lax.axis_index("s")
    g = core * NSUB + sub

    _zero_vmem(zbuf, 104, D)
    for c, x in enumerate((x0, x1, x2)):
        _zero_rows(sub, acc, zbuf)
        plsc.subcore_barrier()

        @pl.loop(0, NFULL)
        def _(j):
            base = pl.multiple_of(g * EPT + j * CH, 8)
            pltpu.sync_copy(ss_s.at[pl.ds(base, CH)], idx_s)
            pltpu.sync_copy(ss_d.at[pl.ds(base, CH)], idx_d)
            pltpu.async_copy(x.at[idx_s], rows_v, sem).wait()
            pltpu.sync_copy(rows_v, acc.at[idx_d], add=True)
        if TAIL:
            base = pl.multiple_of(g * EPT + NFULL * CH, 8)
            pltpu.sync_copy(ss_s.at[pl.ds(base, TAIL)], idx_st)
            pltpu.sync_copy(ss_d.at[pl.ds(base, TAIL)], idx_dt)
            pltpu.async_copy(x.at[idx_st], rows_t, sem).wait()
            pltpu.sync_copy(rows_t, acc.at[idx_dt], add=True)

        plsc.subcore_barrier()
        _rows_sweep(sub, lambda off, n, ci=c: pltpu.sync_copy(
            acc.at[pl.ds(off, n)], out_hbm.at[core, ci, pl.ds(off, n)]))


def _spmm_ss(x0, x1, x2, ss_s, ss_d):
    fn = pl.kernel(
        _sc_spmm_ss,
        out_type=jax.ShapeDtypeStruct((NCORES, 3, NU, D), jnp.float32),
        mesh=_mesh(),
        scratch_types=[
            pltpu.VMEM_SHARED((NU, D), jnp.float32),
            pltpu.VMEM((CH,), jnp.int32),
            pltpu.VMEM((CH,), jnp.int32),
            pltpu.VMEM((TAIL,), jnp.int32),
            pltpu.VMEM((TAIL,), jnp.int32),
            pltpu.VMEM((CH, D), jnp.float32),
            pltpu.VMEM((TAIL, D), jnp.float32),
            pltpu.VMEM((104, D), jnp.float32),
            pltpu.SemaphoreType.DMA,
        ],
    )
    return fn(x0, x1, x2, ss_s, ss_d)


def _sc_spmm_ui(xu, xi, ui_u, ui_i, out_hbm,
                accu, acci, idx_u, idx_i, idx_ut, idx_it,
                rows_v, rows_t, zbuf, sem):
    core = lax.axis_index("c")
    sub = lax.axis_index("s")
    g = core * NSUB + sub

    _zero_vmem(zbuf, 104, D)
    for acc in (accu, acci):
        _zero_rows(sub, acc, zbuf)
    plsc.subcore_barrier()

    @pl.loop(0, NFULL)
    def _(j):
        base = pl.multiple_of(g * EPT + j * CH, 8)
        pltpu.sync_copy(ui_u.at[pl.ds(base, CH)], idx_u)
        pltpu.sync_copy(ui_i.at[pl.ds(base, CH)], idx_i)
        pltpu.async_copy(xu.at[idx_u], rows_v, sem).wait()
        pltpu.sync_copy(rows_v, acci.at[idx_i], add=True)
        pltpu.async_copy(xi.at[idx_i], rows_v, sem).wait()
        pltpu.sync_copy(rows_v, accu.at[idx_u], add=True)
    if TAIL:
        base = pl.multiple_of(g * EPT + NFULL * CH, 8)
        pltpu.sync_copy(ui_u.at[pl.ds(base, TAIL)], idx_ut)
        pltpu.sync_copy(ui_i.at[pl.ds(base, TAIL)], idx_it)
        pltpu.async_copy(xu.at[idx_ut], rows_t, sem).wait()
        pltpu.sync_copy(rows_t, acci.at[idx_it], add=True)
        pltpu.async_copy(xi.at[idx_it], rows_t, sem).wait()
        pltpu.sync_copy(rows_t, accu.at[idx_ut], add=True)

    plsc.subcore_barrier()
    for base_off, acc in ((0, accu), (NU, acci)):
        _rows_sweep(sub, lambda off, n, a=acc, o=base_off: pltpu.sync_copy(
            a.at[pl.ds(off, n)],
            out_hbm.at[core, pl.ds(pl.multiple_of(o + off, 8), n)]))


def _spmm_ui(xu, xi, ui_u, ui_i):
    fn = pl.kernel(
        _sc_spmm_ui,
        out_type=jax.ShapeDtypeStruct((NCORES, NU + NI, D), jnp.float32),
        mesh=_mesh(),
        scratch_types=[
            pltpu.VMEM_SHARED((NU, D), jnp.float32),
            pltpu.VMEM_SHARED((NI, D), jnp.float32),
            pltpu.VMEM((CH,), jnp.int32),
            pltpu.VMEM((CH,), jnp.int32),
            pltpu.VMEM((TAIL,), jnp.int32),
            pltpu.VMEM((TAIL,), jnp.int32),
            pltpu.VMEM((CH, D), jnp.float32),
            pltpu.VMEM((TAIL, D), jnp.float32),
            pltpu.VMEM((104, D), jnp.float32),
            pltpu.SemaphoreType.DMA,
        ],
    )
    return fn(xu, xi, ui_u, ui_i)


def _tc_prep_body(u_ref, gw_ref, gb_ref, degp_ref,
                  x0_ref, x1_ref, x2_ref, acc_ref, r_ref):
    u = u_ref[...]
    degp = degp_ref[...]
    d = degp[0, :, :, 0:1] + degp[1, :, :, 0:1]
    r = lax.rsqrt(d + 1.0)
    r_ref[...] = r
    rs = r[0]
    xr = (x0_ref, x1_ref, x2_ref)
    for c in range(3):
        gate = jax.nn.sigmoid(
            jnp.dot(u, gw_ref[c], preferred_element_type=jnp.float32)
            + gb_ref[c][None, :])
        cur = u * gate
        acc_ref[c] = cur
        xr[c][...] = cur * rs


def _tc_prep(user_emb, gate_W, gate_b, degp):
    sd = jax.ShapeDtypeStruct
    return pl.pallas_call(
        _tc_prep_body,
        out_shape=(sd((NU, D), jnp.float32), sd((NU, D), jnp.float32),
                   sd((NU, D), jnp.float32), sd((3, NU, D), jnp.float32),
                   sd((4, NU, 1), jnp.float32)),
    )(user_emb, gate_W, gate_b, degp)


def _tc_comb_ss_body(need_x, p_ref, accin_ref, r_ref, *outs):
    r = r_ref[...]
    cur = (p_ref[0] + p_ref[1]) * r[1][None]
    acc = accin_ref[...] + cur
    if need_x:
        acc_ref, x0_ref, x1_ref, x2_ref = outs
        rs = r[0]
        x0_ref[...] = cur[0] * rs
        x1_ref[...] = cur[1] * rs
        x2_ref[...] = cur[2] * rs
    else:
        (acc_ref,) = outs
    acc_ref[...] = acc


def _tc_comb_ss(partial, acc_in, r, need_x):
    sd = jax.ShapeDtypeStruct
    outs = (sd((3, NU, D), jnp.float32),)
    if need_x:
        outs = outs + (sd((NU, D), jnp.float32),) * 3
    return pl.pallas_call(
        functools.partial(_tc_comb_ss_body, need_x),
        out_shape=outs,
    )(partial, acc_in, r)


def _tc_fuse_body(acc_ref, r_ref, item_ref, attn_ref,
                  xu_ref, xi_ref, accui_ref):
    r = r_ref[...]
    item = item_ref[...]
    ch = acc_ref[...] * (1.0 / 3.0)
    t = jnp.tanh(ch)
    s = jnp.sum(t * attn_ref[...][None, None, :], axis=-1, keepdims=True)
    m = jnp.max(s, axis=0, keepdims=True)
    e = jnp.exp(s - m)
    w = e / jnp.sum(e, axis=0, keepdims=True)
    uf = jnp.sum(w * ch, axis=0)
    accui_ref[0:NU] = uf
    accui_ref[NU:] = item
    xu_ref[...] = uf * r[2]
    xi_ref[...] = item * r[3]


def _tc_fuse(acc, r, item_emb, attn_W):
    sd = jax.ShapeDtypeStruct
    return pl.pallas_call(
        _tc_fuse_body,
        out_shape=(sd((NU, D), jnp.float32), sd((NI, D), jnp.float32),
                   sd((NU + NI, D), jnp.float32)),
    )(acc, r, item_emb, attn_W)


def _tc_comb_ui_body(final, p_ref, accin_ref, r_ref, *outs):
    r = r_ref[...]
    pp = p_ref[0] + p_ref[1]
    cur_u = pp[0:NU] * r[2]
    cur_i = pp[NU:] * r[3]
    acc_u = accin_ref[0:NU] + cur_u
    acc_i = accin_ref[NU:] + cur_i
    if final:
        (out_ref,) = outs
        out_ref[0:NU] = acc_u * (1.0 / 3.0)
        out_ref[NU:] = acc_i * (1.0 / 3.0)
    else:
        acc_ref, xu_ref, xi_ref = outs
        acc_ref[0:NU] = acc_u
        acc_ref[NU:] = acc_i
        xu_ref[...] = cur_u * r[2]
        xi_ref[...] = cur_i * r[3]


def _tc_comb_ui(partial, acc_in, r, final):
    sd = jax.ShapeDtypeStruct
    if final:
        outs = sd((NU + NI, D), jnp.float32)
    else:
        outs = (sd((NU + NI, D), jnp.float32), sd((NU, D), jnp.float32),
                sd((NI, D), jnp.float32))
    return pl.pallas_call(
        functools.partial(_tc_comb_ui_body, final),
        out_shape=outs,
    )(partial, acc_in, r)


def kernel(user_emb, item_emb, gate_W, gate_b, attn_W,
           ui_edge_index, ss_edge_index):
    ss_s = ss_edge_index[0]
    ss_d = ss_edge_index[1]
    ui_u = ui_edge_index[0]
    ui_i = ui_edge_index[1]

    degp = _degrees(ss_s, ss_d, ui_u, ui_i)
    x0, x1, x2, acc, r = _tc_prep(user_emb, gate_W, gate_b, degp)

    p = _spmm_ss(x0, x1, x2, ss_s, ss_d)
    acc, x0, x1, x2 = _tc_comb_ss(p, acc, r, need_x=True)
    p = _spmm_ss(x0, x1, x2, ss_s, ss_d)
    (acc,) = _tc_comb_ss(p, acc, r, need_x=False)

    xu, xi, acc_ui = _tc_fuse(acc, r, item_emb, attn_W)

    p = _spmm_ui(xu, xi, ui_u, ui_i)
    acc_ui, xu, xi = _tc_comb_ui(p, acc_ui, r, final=False)
    p = _spmm_ui(xu, xi, ui_u, ui_i)
    return _tc_comb_ui(p, acc_ui, r, final=True)
